# baseline (speedup 1.0000x reference)
import functools

import jax
import jax.numpy as jnp
from jax import lax
from jax.experimental import pallas as pl
from jax.experimental.pallas import tpu as pltpu
from jax.experimental.pallas import tpu_sc as plsc

_OHEM_RATIO = 3.0
_SMOOTH = 1e-06
_SENT = -2147483648
_N_IMG = 8
_HW = 512 * 512
_CHUNK = _HW // 16
_VECS = _CHUNK // 16


def _scan_hist(merged_v, k):
    def body(i, carry):
        acc, running = carry
        j = 127 - i
        h = merged_v[pl.ds(j * 16, 16)]
        tot = jnp.sum(h)
        pre = plsc.cumsum(h)
        suf_inc = (tot - pre) + h
        t_inc = running + suf_inc
        t_exc = t_inc - h
        idx = j * 16 + lax.iota(jnp.int32, 16)
        packed = jnp.where(t_inc >= k, (idx << 19) + t_exc, -1)
        return jnp.maximum(acc, packed), running + tot

    acc, _ = lax.fori_loop(
        0, 128, body, (jnp.full((16,), -1, jnp.int32), jnp.int32(0))
    )
    p = jnp.max(acc)
    return p >> 19, p & ((1 << 19) - 1)


def _sc_body(preds_hbm, tgts_hbm, eff_hbm, out_hbm,
             key_v, tgt_v, eff_v, hist_v, hist2_v, merged2_v, merged_v,
             zero_v, zc_v, cnt_v, mcnt_v,
             idx16_v, outvec_v, sharedH_v, sharedC_v):
    cid = lax.axis_index("c")
    sid = lax.axis_index("s")
    off = sid * _CHUNK

    iota = lax.iota(jnp.int32, 16)
    ones = jnp.ones((16,), jnp.int32)
    zeros16 = jnp.zeros((16,), jnp.int32)

    for r in range(16):
        for c in range(8):
            zero_v[r, pl.ds(c * 16, 16)] = zeros16
        zc_v[r, :] = zeros16
    idx16_v[...] = iota

    def zero_hist():
        for r in range(128):
            hist_v[pl.ds(r * 16, 16)] = zeros16

    def pack_hist():
        for i in range(128):
            hist2_v[i // 8, pl.ds((i % 8) * 16, 16)] = hist_v[pl.ds(i * 16, 16)]

    def unpack_merged():
        for i in range(128):
            merged_v[pl.ds(i * 16, 16)] = merged2_v[i // 8, pl.ds((i % 8) * 16, 16)]

    def merge_hist():
        pack_hist()
        pltpu.sync_copy(hist2_v, sharedH_v.at[idx16_v], add=True)

    def img_body(t, _):
        img = 2 * t + cid
        pltpu.sync_copy(preds_hbm.at[img, pl.ds(off, _CHUNK)], key_v)
        pltpu.sync_copy(tgts_hbm.at[img, pl.ds(off, _CHUNK)], tgt_v)
        pltpu.sync_copy(eff_hbm.at[img, pl.ds(off, _CHUNK)], eff_v)

        zero_hist()
        for r in range(16):
            cnt_v[r, :] = zeros16

        def p1(j, carry):
            accp, accn = carry
            sl = pl.ds(j * 16, 16)
            v = key_v[sl]
            tt = tgt_v[sl]
            ee = eff_v[sl]
            neg = tt <= 0.5
            key = jnp.where(v >= 0, v, v ^ 0x7FFFFFFF)
            mk = jnp.where(neg, key, _SENT)
            key_v[sl] = mk
            b1 = (mk >> 21) + 1024
            plsc.addupdate_scatter(hist_v, [b1], ones, mask=neg)
            accp = accp + jnp.where((tt > 0.5) & (ee > 0.5), 1, 0)
            accn = accn + jnp.where(neg, 1, 0)
            return accp, accn

        accp, accn = lax.fori_loop(0, _VECS, p1, (zeros16, zeros16))
        cnt_v[0, :] = accp
        cnt_v[1, :] = accn

        plsc.subcore_barrier()

        @pl.when(sid == 0)
        def _():
            pltpu.sync_copy(zero_v, sharedH_v)
            pltpu.sync_copy(zc_v, sharedC_v)

        plsc.subcore_barrier()
        merge_hist()
        pltpu.sync_copy(cnt_v, sharedC_v.at[idx16_v], add=True)
        plsc.subcore_barrier()
        pltpu.sync_copy(sharedH_v, merged2_v)
        pltpu.sync_copy(sharedC_v, mcnt_v)
        unpack_merged()

        pos_num = jnp.sum(mcnt_v[0, :])
        neg_tot = jnp.sum(mcnt_v[1, :])
        k1 = jnp.minimum(3 * pos_num, neg_tot)

        b1_bin, tex1 = _scan_hist(merged_v, k1)
        k2 = k1 - tex1

        zero_hist()

        def p2(j, c):
            sl = pl.ds(j * 16, 16)
            mk = key_v[sl]
            match = ((mk >> 21) + 1024 == b1_bin) & (mk != _SENT)
            b2 = (mk >> 10) & 0x7FF
            plsc.addupdate_scatter(hist_v, [b2], ones, mask=match)
            return c

        lax.fori_loop(0, _VECS, p2, 0)
        plsc.subcore_barrier()

        @pl.when(sid == 0)
        def _():
            pltpu.sync_copy(zero_v, sharedH_v)

        plsc.subcore_barrier()
        merge_hist()
        plsc.subcore_barrier()
        pltpu.sync_copy(sharedH_v, merged2_v)
        unpack_merged()

        b2_bin, tex2 = _scan_hist(merged_v, k2)
        k3 = k2 - tex2
        pre22 = ((b1_bin - 1024) << 11) + b2_bin

        zero_hist()

        def p3(j, c):
            sl = pl.ds(j * 16, 16)
            mk = key_v[sl]
            match = ((mk >> 10) == pre22) & (mk != _SENT)
            b3 = mk & 0x3FF
            plsc.addupdate_scatter(hist_v, [b3], ones, mask=match)
            return c

        lax.fori_loop(0, _VECS, p3, 0)
        plsc.subcore_barrier()

        @pl.when(sid == 0)
        def _():
            pltpu.sync_copy(zero_v, sharedH_v)

        plsc.subcore_barrier()
        merge_hist()
        plsc.subcore_barrier()
        pltpu.sync_copy(sharedH_v, merged2_v)
        unpack_merged()

        b3_bin, _tex3 = _scan_hist(merged_v, k3)
        keystar = (pre22 << 10) + b3_bin
        lbits = jnp.where(keystar >= 0, keystar, keystar ^ 0x7FFFFFFF)

        @pl.when(sid == 0)
        def _():
            ov = jnp.where(
                iota == 0,
                lbits,
                jnp.where(iota == 1, pos_num, jnp.where(iota == 2, k1, 0)),
            )
            outvec_v[...] = ov
            pltpu.sync_copy(outvec_v, out_hbm.at[img])

        plsc.subcore_barrier()
        return 0

    lax.fori_loop(0, _N_IMG // 2, img_body, 0)


def _sc_select(preds, targets, effective_maps):
    preds_i = lax.bitcast_convert_type(preds, jnp.int32).reshape(_N_IMG, -1)
    tgts = targets.reshape(_N_IMG, -1)
    eff = effective_maps.reshape(_N_IMG, -1)
    mesh = plsc.VectorSubcoreMesh(core_axis_name="c", subcore_axis_name="s")
    fn = pl.kernel(
        _sc_body,
        out_type=jax.ShapeDtypeStruct((_N_IMG, 16), jnp.int32),
        mesh=mesh,
        scratch_types=[
            pltpu.VMEM((_CHUNK,), jnp.int32),
            pltpu.VMEM((_CHUNK,), jnp.float32),
            pltpu.VMEM((_CHUNK,), jnp.float32),
            pltpu.VMEM((2048,), jnp.int32),
            pltpu.VMEM((16, 128), jnp.int32),
            pltpu.VMEM((16, 128), jnp.int32),
            pltpu.VMEM((2048,), jnp.int32),
            pltpu.VMEM((16, 128), jnp.int32),
            pltpu.VMEM((16, 16), jnp.int32),
            pltpu.VMEM((16, 16), jnp.int32),
            pltpu.VMEM((16, 16), jnp.int32),
            pltpu.VMEM((16,), jnp.int32),
            pltpu.VMEM((16,), jnp.int32),
            pltpu.VMEM_SHARED((16, 128), jnp.int32),
            pltpu.VMEM_SHARED((16, 16), jnp.int32),
        ],
        compiler_params=pltpu.CompilerParams(needs_layout_passes=False),
    )
    return fn(preds_i, tgts, eff)


def _loss_body(pt_ref, tt_ref, pk_ref, tk_ref, eff_ref, thr_ref,
               text_ref, kern_ref):
    logits_t = pt_ref[0, 0]
    tt = tt_ref[0, 0]
    eff = eff_ref[0]
    pred = jax.nn.sigmoid(logits_t)

    pos = tt > 0.5
    effg = eff > 0.5

    lbits = thr_ref[0, 0, 0]
    pos_num = thr_ref[0, 0, 1]
    neg_num = thr_ref[0, 0, 2]

    tlogit = lax.bitcast_convert_type(jnp.full((1, 1), lbits, jnp.int32),
                                      jnp.float32)
    thr = jax.nn.sigmoid(tlogit)

    sel = ((pred >= thr) | pos) & effg
    cond = (pos_num == 0) | (neg_num == 0)
    sel_f = jnp.where(
        cond,
        (eff != 0.0).astype(jnp.float32),
        sel.astype(jnp.float32),
    )

    t_f = pos.astype(jnp.float32) * sel_f
    p_f = pred * sel_f
    pg = jnp.sum(p_f * t_f)
    p2 = jnp.sum(p_f * p_f)
    g2 = jnp.sum(t_f * t_f)
    text_loss = 1.0 - (2.0 * pg + _SMOOTH) / (p2 + g2 + _SMOOTH)

    pred_k = jax.nn.sigmoid(pk_ref[0, 0])
    sel2 = ((pred > 0.5) & effg).astype(jnp.float32)
    tk_f = (tk_ref[0, 0] > 0.5).astype(jnp.float32) * sel2
    pk_f = pred_k * sel2
    pg2 = jnp.sum(pk_f * tk_f)
    p2b = jnp.sum(pk_f * pk_f)
    g2b = jnp.sum(tk_f * tk_f)
    kernel_loss = 1.0 - (2.0 * pg2 + _SMOOTH) / (p2b + g2b + _SMOOTH)

    text_ref[0, 0, :] = jnp.full((128,), text_loss, dtype=jnp.float32)
    kern_ref[0, 0, :] = jnp.full((128,), kernel_loss, dtype=jnp.float32)


def kernel(preds, targets, effective_maps):
    n, _, h, w = preds.shape
    thr = _sc_select(preds, targets, effective_maps).reshape(n, 1, 16)

    img_spec = pl.BlockSpec((1, h, w), lambda i: (i, 0, 0))
    ch0_spec = pl.BlockSpec((1, 1, h, w), lambda i: (i, 0, 0, 0))
    ch1_spec = pl.BlockSpec((1, 1, h, w), lambda i: (i, 1, 0, 0))
    thr_spec = pl.BlockSpec((1, 1, 16), lambda i: (i, 0, 0),
                            memory_space=pltpu.SMEM)
    out_spec = pl.BlockSpec((1, 1, 128), lambda i: (i, 0, 0))
    text, kern = pl.pallas_call(
        _loss_body,
        grid=(n,),
        in_specs=[ch0_spec, ch0_spec, ch1_spec, ch1_spec, img_spec, thr_spec],
        out_specs=[out_spec, out_spec],
        out_shape=[
            jax.ShapeDtypeStruct((n, 1, 128), jnp.float32),
            jax.ShapeDtypeStruct((n, 1, 128), jnp.float32),
        ],
        compiler_params=pltpu.CompilerParams(
            dimension_semantics=("arbitrary",),
        ),
    )(preds, targets, preds, targets, effective_maps, thr)
    return text[:, 0, 0], kern[:, 0, 0]


# speedup vs baseline: 1.2913x; 1.2913x over previous
import functools

import jax
import jax.numpy as jnp
from jax import lax
from jax.experimental import pallas as pl
from jax.experimental.pallas import tpu as pltpu
from jax.experimental.pallas import tpu_sc as plsc

_OHEM_RATIO = 3.0
_SMOOTH = 1e-06
_SENT = -2147483648
_N_IMG = 8
_HW = 512 * 512
_CHUNK = _HW // 16
_VECS = _CHUNK // 16


def _scan_hist(merged_v, k):
    def body(i, carry):
        acc, running = carry
        j = 127 - i
        h = merged_v[pl.ds(j * 16, 16)]
        tot = jnp.sum(h)
        pre = plsc.cumsum(h)
        suf_inc = (tot - pre) + h
        t_inc = running + suf_inc
        t_exc = t_inc - h
        idx = j * 16 + lax.iota(jnp.int32, 16)
        packed = jnp.where(t_inc >= k, (idx << 19) + t_exc, -1)
        return jnp.maximum(acc, packed), running + tot

    acc, _ = plsc.parallel_loop(
        0, 128, carry=(jnp.full((16,), -1, jnp.int32), jnp.int32(0)), unroll=4
    )(body)
    p = jnp.max(acc)
    return p >> 19, p & ((1 << 19) - 1)


def _sc_body(preds_hbm, tgts_hbm, eff_hbm, out_hbm,
             key_v, tgt_v, eff_v, hist_v, hist2_v, merged2_v, merged_v,
             zero_v, zc_v, cnt_v, mcnt_v,
             idx16_v, outvec_v, sharedH_v, sharedC_v):
    cid = lax.axis_index("c")
    sid = lax.axis_index("s")
    off = sid * _CHUNK

    iota = lax.iota(jnp.int32, 16)
    ones = jnp.ones((16,), jnp.int32)
    zeros16 = jnp.zeros((16,), jnp.int32)

    for r in range(16):
        for c in range(8):
            zero_v[r, pl.ds(c * 16, 16)] = zeros16
        zc_v[r, :] = zeros16
    idx16_v[...] = iota

    def zero_hist():
        for r in range(128):
            hist_v[pl.ds(r * 16, 16)] = zeros16

    def pack_hist():
        for i in range(128):
            hist2_v[i // 8, pl.ds((i % 8) * 16, 16)] = hist_v[pl.ds(i * 16, 16)]

    def unpack_merged():
        for i in range(128):
            merged_v[pl.ds(i * 16, 16)] = merged2_v[i // 8, pl.ds((i % 8) * 16, 16)]

    def merge_hist():
        pack_hist()
        pltpu.sync_copy(hist2_v, sharedH_v.at[idx16_v], add=True)

    def img_body(t, _):
        img = 2 * t + cid
        pltpu.sync_copy(preds_hbm.at[img, pl.ds(off, _CHUNK)], key_v)
        pltpu.sync_copy(tgts_hbm.at[img, pl.ds(off, _CHUNK)], tgt_v)
        pltpu.sync_copy(eff_hbm.at[img, pl.ds(off, _CHUNK)], eff_v)

        zero_hist()
        for r in range(16):
            cnt_v[r, :] = zeros16

        def p1(j, carry):
            accp, accn = carry
            sl = pl.ds(j * 16, 16)
            v = key_v[sl]
            tt = tgt_v[sl]
            ee = eff_v[sl]
            neg = tt <= 0.5
            key = jnp.where(v >= 0, v, v ^ 0x7FFFFFFF)
            mk = jnp.where(neg, key, _SENT)
            key_v[sl] = mk
            b1 = (mk >> 21) + 1024
            plsc.addupdate_scatter(hist_v, [b1], ones, mask=neg)
            accp = accp + jnp.where((tt > 0.5) & (ee > 0.5), 1, 0)
            accn = accn + jnp.where(neg, 1, 0)
            return accp, accn

        accp, accn = plsc.parallel_loop(0, _VECS, carry=(zeros16, zeros16), unroll=8)(p1)
        cnt_v[0, :] = accp
        cnt_v[1, :] = accn

        plsc.subcore_barrier()

        @pl.when(sid == 0)
        def _():
            pltpu.sync_copy(zero_v, sharedH_v)
            pltpu.sync_copy(zc_v, sharedC_v)

        plsc.subcore_barrier()
        merge_hist()
        pltpu.sync_copy(cnt_v, sharedC_v.at[idx16_v], add=True)
        plsc.subcore_barrier()
        pltpu.sync_copy(sharedH_v, merged2_v)
        pltpu.sync_copy(sharedC_v, mcnt_v)
        unpack_merged()

        pos_num = jnp.sum(mcnt_v[0, :])
        neg_tot = jnp.sum(mcnt_v[1, :])
        k1 = jnp.minimum(3 * pos_num, neg_tot)

        b1_bin, tex1 = _scan_hist(merged_v, k1)
        k2 = k1 - tex1

        zero_hist()

        def p2(j):
            sl = pl.ds(j * 16, 16)
            mk = key_v[sl]
            match = ((mk >> 21) + 1024 == b1_bin) & (mk != _SENT)
            b2 = (mk >> 10) & 0x7FF
            plsc.addupdate_scatter(hist_v, [b2], ones, mask=match)

        plsc.parallel_loop(0, _VECS, unroll=8)(p2)
        plsc.subcore_barrier()

        @pl.when(sid == 0)
        def _():
            pltpu.sync_copy(zero_v, sharedH_v)

        plsc.subcore_barrier()
        merge_hist()
        plsc.subcore_barrier()
        pltpu.sync_copy(sharedH_v, merged2_v)
        unpack_merged()

        b2_bin, tex2 = _scan_hist(merged_v, k2)
        k3 = k2 - tex2
        pre22 = ((b1_bin - 1024) << 11) + b2_bin

        zero_hist()

        def p3(j):
            sl = pl.ds(j * 16, 16)
            mk = key_v[sl]
            match = ((mk >> 10) == pre22) & (mk != _SENT)
            b3 = mk & 0x3FF
            plsc.addupdate_scatter(hist_v, [b3], ones, mask=match)

        plsc.parallel_loop(0, _VECS, unroll=8)(p3)
        plsc.subcore_barrier()

        @pl.when(sid == 0)
        def _():
            pltpu.sync_copy(zero_v, sharedH_v)

        plsc.subcore_barrier()
        merge_hist()
        plsc.subcore_barrier()
        pltpu.sync_copy(sharedH_v, merged2_v)
        unpack_merged()

        b3_bin, _tex3 = _scan_hist(merged_v, k3)
        keystar = (pre22 << 10) + b3_bin
        lbits = jnp.where(keystar >= 0, keystar, keystar ^ 0x7FFFFFFF)

        @pl.when(sid == 0)
        def _():
            ov = jnp.where(
                iota == 0,
                lbits,
                jnp.where(iota == 1, pos_num, jnp.where(iota == 2, k1, 0)),
            )
            outvec_v[...] = ov
            pltpu.sync_copy(outvec_v, out_hbm.at[img])

        plsc.subcore_barrier()
        return 0

    lax.fori_loop(0, _N_IMG // 2, img_body, 0)


def _sc_select(preds, targets, effective_maps):
    preds_i = lax.bitcast_convert_type(preds, jnp.int32).reshape(_N_IMG, -1)
    tgts = targets.reshape(_N_IMG, -1)
    eff = effective_maps.reshape(_N_IMG, -1)
    mesh = plsc.VectorSubcoreMesh(core_axis_name="c", subcore_axis_name="s")
    fn = pl.kernel(
        _sc_body,
        out_type=jax.ShapeDtypeStruct((_N_IMG, 16), jnp.int32),
        mesh=mesh,
        scratch_types=[
            pltpu.VMEM((_CHUNK,), jnp.int32),
            pltpu.VMEM((_CHUNK,), jnp.float32),
            pltpu.VMEM((_CHUNK,), jnp.float32),
            pltpu.VMEM((2048,), jnp.int32),
            pltpu.VMEM((16, 128), jnp.int32),
            pltpu.VMEM((16, 128), jnp.int32),
            pltpu.VMEM((2048,), jnp.int32),
            pltpu.VMEM((16, 128), jnp.int32),
            pltpu.VMEM((16, 16), jnp.int32),
            pltpu.VMEM((16, 16), jnp.int32),
            pltpu.VMEM((16, 16), jnp.int32),
            pltpu.VMEM((16,), jnp.int32),
            pltpu.VMEM((16,), jnp.int32),
            pltpu.VMEM_SHARED((16, 128), jnp.int32),
            pltpu.VMEM_SHARED((16, 16), jnp.int32),
        ],
        compiler_params=pltpu.CompilerParams(needs_layout_passes=False),
    )
    return fn(preds_i, tgts, eff)


def _loss_body(pt_ref, tt_ref, pk_ref, tk_ref, eff_ref, thr_ref,
               text_ref, kern_ref):
    logits_t = pt_ref[0, 0]
    tt = tt_ref[0, 0]
    eff = eff_ref[0]
    pred = jax.nn.sigmoid(logits_t)

    pos = tt > 0.5
    effg = eff > 0.5

    lbits = thr_ref[0, 0, 0]
    pos_num = thr_ref[0, 0, 1]
    neg_num = thr_ref[0, 0, 2]

    tlogit = lax.bitcast_convert_type(jnp.full((1, 1), lbits, jnp.int32),
                                      jnp.float32)
    thr = jax.nn.sigmoid(tlogit)

    sel = ((pred >= thr) | pos) & effg
    cond = (pos_num == 0) | (neg_num == 0)
    sel_f = jnp.where(
        cond,
        (eff != 0.0).astype(jnp.float32),
        sel.astype(jnp.float32),
    )

    t_f = pos.astype(jnp.float32) * sel_f
    p_f = pred * sel_f
    pg = jnp.sum(p_f * t_f)
    p2 = jnp.sum(p_f * p_f)
    g2 = jnp.sum(t_f * t_f)
    text_loss = 1.0 - (2.0 * pg + _SMOOTH) / (p2 + g2 + _SMOOTH)

    pred_k = jax.nn.sigmoid(pk_ref[0, 0])
    sel2 = ((pred > 0.5) & effg).astype(jnp.float32)
    tk_f = (tk_ref[0, 0] > 0.5).astype(jnp.float32) * sel2
    pk_f = pred_k * sel2
    pg2 = jnp.sum(pk_f * tk_f)
    p2b = jnp.sum(pk_f * pk_f)
    g2b = jnp.sum(tk_f * tk_f)
    kernel_loss = 1.0 - (2.0 * pg2 + _SMOOTH) / (p2b + g2b + _SMOOTH)

    text_ref[0, 0, :] = jnp.full((128,), text_loss, dtype=jnp.float32)
    kern_ref[0, 0, :] = jnp.full((128,), kernel_loss, dtype=jnp.float32)


def kernel(preds, targets, effective_maps):
    n, _, h, w = preds.shape
    thr = _sc_select(preds, targets, effective_maps).reshape(n, 1, 16)

    img_spec = pl.BlockSpec((1, h, w), lambda i: (i, 0, 0))
    ch0_spec = pl.BlockSpec((1, 1, h, w), lambda i: (i, 0, 0, 0))
    ch1_spec = pl.BlockSpec((1, 1, h, w), lambda i: (i, 1, 0, 0))
    thr_spec = pl.BlockSpec((1, 1, 16), lambda i: (i, 0, 0),
                            memory_space=pltpu.SMEM)
    out_spec = pl.BlockSpec((1, 1, 128), lambda i: (i, 0, 0))
    text, kern = pl.pallas_call(
        _loss_body,
        grid=(n,),
        in_specs=[ch0_spec, ch0_spec, ch1_spec, ch1_spec, img_spec, thr_spec],
        out_specs=[out_spec, out_spec],
        out_shape=[
            jax.ShapeDtypeStruct((n, 1, 128), jnp.float32),
            jax.ShapeDtypeStruct((n, 1, 128), jnp.float32),
        ],
        compiler_params=pltpu.CompilerParams(
            dimension_semantics=("arbitrary",),
        ),
    )(preds, targets, preds, targets, effective_maps, thr)
    return text[:, 0, 0], kern[:, 0, 0]


# speedup vs baseline: 1.6448x; 1.2737x over previous
import functools

import jax
import jax.numpy as jnp
from jax import lax
from jax.experimental import pallas as pl
from jax.experimental.pallas import tpu as pltpu
from jax.experimental.pallas import tpu_sc as plsc

_OHEM_RATIO = 3.0
_SMOOTH = 1e-06
_SENT = -2147483648
_N_IMG = 8
_HW = 512 * 512
_CHUNK = _HW // 16
_VECS = _CHUNK // 16


def _scan_hist(merged_v, k):
    def body(i, carry):
        acc, running = carry
        j = 127 - i
        h = merged_v[pl.ds(j * 16, 16)]
        tot = jnp.sum(h)
        pre = plsc.cumsum(h)
        suf_inc = (tot - pre) + h
        t_inc = running + suf_inc
        t_exc = t_inc - h
        idx = j * 16 + lax.iota(jnp.int32, 16)
        packed = jnp.where(t_inc >= k, (idx << 19) + t_exc, -1)
        return jnp.maximum(acc, packed), running + tot

    acc, _ = plsc.parallel_loop(
        0, 128, carry=(jnp.full((16,), -1, jnp.int32), jnp.int32(0)), unroll=4
    )(body)
    p = jnp.max(acc)
    return p >> 19, p & ((1 << 19) - 1)


def _sc_body(preds_hbm, tgts_hbm, eff_hbm, out_hbm,
             key_v, tgt_v, eff_v, hist_v, hist2_v, merged2_v, merged_v,
             zero_v, zc_v, cnt_v, mcnt_v,
             idx16_v, outvec_v, sharedH_v, sharedC_v):
    cid = lax.axis_index("c")
    sid = lax.axis_index("s")
    off = sid * _CHUNK

    iota = lax.iota(jnp.int32, 16)
    ones = jnp.ones((16,), jnp.int32)
    zeros16 = jnp.zeros((16,), jnp.int32)

    for r in range(16):
        for c in range(8):
            zero_v[r, pl.ds(c * 16, 16)] = zeros16
        zc_v[r, :] = zeros16
    idx16_v[...] = iota

    def zero_hist():
        for r in range(128):
            hist_v[pl.ds(r * 16, 16)] = zeros16

    def pack_hist():
        for i in range(128):
            hist2_v[i // 8, pl.ds((i % 8) * 16, 16)] = hist_v[pl.ds(i * 16, 16)]

    def unpack_merged():
        for i in range(128):
            merged_v[pl.ds(i * 16, 16)] = merged2_v[i // 8, pl.ds((i % 8) * 16, 16)]

    def merge_hist():
        pack_hist()
        pltpu.sync_copy(hist2_v, sharedH_v.at[idx16_v], add=True)

    def img_body(t, _):
        img = 2 * t + cid
        pltpu.sync_copy(preds_hbm.at[img, pl.ds(off, _CHUNK)], key_v)
        pltpu.sync_copy(tgts_hbm.at[img, pl.ds(off, _CHUNK)], tgt_v)
        pltpu.sync_copy(eff_hbm.at[img, pl.ds(off, _CHUNK)], eff_v)

        zero_hist()
        for r in range(16):
            cnt_v[r, :] = zeros16

        def p1(j, carry):
            accp, accn = carry
            sl = pl.ds(j * 16, 16)
            v = key_v[sl]
            tt = tgt_v[sl]
            ee = eff_v[sl]
            neg = tt <= 0.5
            key = jnp.where(v >= 0, v, v ^ 0x7FFFFFFF)
            mk = jnp.where(neg, key, _SENT)
            key_v[sl] = mk
            b1 = (mk >> 21) + 1024
            plsc.addupdate_scatter(hist_v, [b1], ones, mask=neg)
            accp = accp + jnp.where((tt > 0.5) & (ee > 0.5), 1, 0)
            accn = accn + jnp.where(neg, 1, 0)
            return accp, accn

        accp, accn = plsc.parallel_loop(0, _VECS, carry=(zeros16, zeros16), unroll=8)(p1)
        cnt_v[0, :] = accp
        cnt_v[1, :] = accn

        plsc.subcore_barrier()

        @pl.when(sid == 0)
        def _():
            pltpu.sync_copy(zero_v, sharedH_v)
            pltpu.sync_copy(zc_v, sharedC_v)

        plsc.subcore_barrier()
        merge_hist()
        pltpu.sync_copy(cnt_v, sharedC_v.at[idx16_v], add=True)
        plsc.subcore_barrier()
        pltpu.sync_copy(sharedH_v, merged2_v)
        pltpu.sync_copy(sharedC_v, mcnt_v)
        unpack_merged()

        pos_num = jnp.sum(mcnt_v[0, :])
        neg_tot = jnp.sum(mcnt_v[1, :])
        k1 = jnp.minimum(3 * pos_num, neg_tot)

        b1_bin, tex1 = _scan_hist(merged_v, k1)
        k2 = k1 - tex1

        zero_hist()

        def p2(j):
            sl = pl.ds(j * 16, 16)
            mk = key_v[sl]
            match = ((mk >> 21) + 1024 == b1_bin) & (mk != _SENT)
            b2 = (mk >> 10) & 0x7FF
            plsc.addupdate_scatter(hist_v, [b2], ones, mask=match)

        plsc.parallel_loop(0, _VECS, unroll=8)(p2)
        plsc.subcore_barrier()

        @pl.when(sid == 0)
        def _():
            pltpu.sync_copy(zero_v, sharedH_v)

        plsc.subcore_barrier()
        merge_hist()
        plsc.subcore_barrier()
        pltpu.sync_copy(sharedH_v, merged2_v)
        unpack_merged()

        b2_bin, tex2 = _scan_hist(merged_v, k2)
        k3 = k2 - tex2
        pre22 = ((b1_bin - 1024) << 11) + b2_bin

        zero_hist()

        def p3(j):
            sl = pl.ds(j * 16, 16)
            mk = key_v[sl]
            match = ((mk >> 10) == pre22) & (mk != _SENT)
            b3 = mk & 0x3FF
            plsc.addupdate_scatter(hist_v, [b3], ones, mask=match)

        plsc.parallel_loop(0, _VECS, unroll=8)(p3)
        plsc.subcore_barrier()

        @pl.when(sid == 0)
        def _():
            pltpu.sync_copy(zero_v, sharedH_v)

        plsc.subcore_barrier()
        merge_hist()
        plsc.subcore_barrier()
        pltpu.sync_copy(sharedH_v, merged2_v)
        unpack_merged()

        b3_bin, _tex3 = _scan_hist(merged_v, k3)
        keystar = (pre22 << 10) + b3_bin
        lbits = jnp.where(keystar >= 0, keystar, keystar ^ 0x7FFFFFFF)

        @pl.when(sid == 0)
        def _():
            ov = jnp.where(
                iota == 0,
                lbits,
                jnp.where(iota == 1, pos_num, jnp.where(iota == 2, k1, 0)),
            )
            outvec_v[...] = ov
            pltpu.sync_copy(outvec_v, out_hbm.at[img])

        plsc.subcore_barrier()
        return 0

    lax.fori_loop(0, _N_IMG // 2, img_body, 0)


def _sc_select(preds, targets, effective_maps):
    preds_i = lax.bitcast_convert_type(preds[:, 0, :, :], jnp.int32).reshape(
        _N_IMG, -1
    )
    tgts = targets[:, 0, :, :].reshape(_N_IMG, -1)
    eff = effective_maps.reshape(_N_IMG, -1)
    mesh = plsc.VectorSubcoreMesh(core_axis_name="c", subcore_axis_name="s")
    fn = pl.kernel(
        _sc_body,
        out_type=jax.ShapeDtypeStruct((_N_IMG, 16), jnp.int32),
        mesh=mesh,
        scratch_types=[
            pltpu.VMEM((_CHUNK,), jnp.int32),
            pltpu.VMEM((_CHUNK,), jnp.float32),
            pltpu.VMEM((_CHUNK,), jnp.float32),
            pltpu.VMEM((2048,), jnp.int32),
            pltpu.VMEM((16, 128), jnp.int32),
            pltpu.VMEM((16, 128), jnp.int32),
            pltpu.VMEM((2048,), jnp.int32),
            pltpu.VMEM((16, 128), jnp.int32),
            pltpu.VMEM((16, 16), jnp.int32),
            pltpu.VMEM((16, 16), jnp.int32),
            pltpu.VMEM((16, 16), jnp.int32),
            pltpu.VMEM((16,), jnp.int32),
            pltpu.VMEM((16,), jnp.int32),
            pltpu.VMEM_SHARED((16, 128), jnp.int32),
            pltpu.VMEM_SHARED((16, 16), jnp.int32),
        ],
        compiler_params=pltpu.CompilerParams(needs_layout_passes=False),
    )
    return fn(preds_i, tgts, eff)


def _loss_body(pt_ref, tt_ref, pk_ref, tk_ref, eff_ref, thr_ref,
               text_ref, kern_ref):
    logits_t = pt_ref[0, 0]
    tt = tt_ref[0, 0]
    eff = eff_ref[0]
    pred = jax.nn.sigmoid(logits_t)

    pos = tt > 0.5
    effg = eff > 0.5

    lbits = thr_ref[0, 0, 0]
    pos_num = thr_ref[0, 0, 1]
    neg_num = thr_ref[0, 0, 2]

    tlogit = lax.bitcast_convert_type(jnp.full((1, 1), lbits, jnp.int32),
                                      jnp.float32)
    thr = jax.nn.sigmoid(tlogit)

    sel = ((pred >= thr) | pos) & effg
    cond = (pos_num == 0) | (neg_num == 0)
    sel_f = jnp.where(
        cond,
        (eff != 0.0).astype(jnp.float32),
        sel.astype(jnp.float32),
    )

    t_f = pos.astype(jnp.float32) * sel_f
    p_f = pred * sel_f
    pg = jnp.sum(p_f * t_f)
    p2 = jnp.sum(p_f * p_f)
    g2 = jnp.sum(t_f * t_f)
    text_loss = 1.0 - (2.0 * pg + _SMOOTH) / (p2 + g2 + _SMOOTH)

    pred_k = jax.nn.sigmoid(pk_ref[0, 0])
    sel2 = ((pred > 0.5) & effg).astype(jnp.float32)
    tk_f = (tk_ref[0, 0] > 0.5).astype(jnp.float32) * sel2
    pk_f = pred_k * sel2
    pg2 = jnp.sum(pk_f * tk_f)
    p2b = jnp.sum(pk_f * pk_f)
    g2b = jnp.sum(tk_f * tk_f)
    kernel_loss = 1.0 - (2.0 * pg2 + _SMOOTH) / (p2b + g2b + _SMOOTH)

    text_ref[0, 0, :] = jnp.full((128,), text_loss, dtype=jnp.float32)
    kern_ref[0, 0, :] = jnp.full((128,), kernel_loss, dtype=jnp.float32)


def kernel(preds, targets, effective_maps):
    n, _, h, w = preds.shape
    thr = _sc_select(preds, targets, effective_maps).reshape(n, 1, 16)

    img_spec = pl.BlockSpec((1, h, w), lambda i: (i, 0, 0))
    ch0_spec = pl.BlockSpec((1, 1, h, w), lambda i: (i, 0, 0, 0))
    ch1_spec = pl.BlockSpec((1, 1, h, w), lambda i: (i, 1, 0, 0))
    thr_spec = pl.BlockSpec((1, 1, 16), lambda i: (i, 0, 0),
                            memory_space=pltpu.SMEM)
    out_spec = pl.BlockSpec((1, 1, 128), lambda i: (i, 0, 0))
    text, kern = pl.pallas_call(
        _loss_body,
        grid=(n,),
        in_specs=[ch0_spec, ch0_spec, ch1_spec, ch1_spec, img_spec, thr_spec],
        out_specs=[out_spec, out_spec],
        out_shape=[
            jax.ShapeDtypeStruct((n, 1, 128), jnp.float32),
            jax.ShapeDtypeStruct((n, 1, 128), jnp.float32),
        ],
        compiler_params=pltpu.CompilerParams(
            dimension_semantics=("arbitrary",),
        ),
    )(preds, targets, preds, targets, effective_maps, thr)
    return text[:, 0, 0], kern[:, 0, 0]


# speedup vs baseline: 3.5787x; 2.1758x over previous
import jax
import jax.numpy as jnp
from jax import lax
from jax.experimental import pallas as pl
from jax.experimental.pallas import tpu as pltpu

_OHEM_RATIO = 3.0
_SMOOTH = 1e-06
_ONE_BITS = 0x3F800000


def _loss_body(pt_ref, tt_ref, pk_ref, tk_ref, eff_ref, text_ref, kern_ref,
               mb_ref):
    logits_t = pt_ref[0, 0]
    tt = tt_ref[0, 0]
    eff = eff_ref[0]
    pred = jax.nn.sigmoid(logits_t)

    pos = tt > 0.5
    neg = jnp.logical_not(pos)
    effg = eff > 0.5

    pos_num = jnp.sum(
        jnp.where(pos & effg, 1, 0), axis=(0, 1), keepdims=True
    )
    neg_total = jnp.sum(jnp.where(neg, 1, 0), axis=(0, 1), keepdims=True)
    neg_num = jnp.minimum(
        pos_num.astype(jnp.float32) * _OHEM_RATIO,
        neg_total.astype(jnp.float32),
    ).astype(jnp.int32)

    bits = lax.bitcast_convert_type(pred, jnp.int32)
    mbits = jnp.where(neg, bits, -1)

    mb_ref[...] = mbits

    def step(_, lohi):
        lo, hi = lohi
        s = (hi - lo + 4) // 4
        m1 = lo + s
        m2 = lo + 2 * s
        m3 = lo + 3 * s
        acc1 = jnp.zeros((8, 512), jnp.int32)
        acc2 = jnp.zeros((8, 512), jnp.int32)
        acc3 = jnp.zeros((8, 512), jnp.int32)
        for r in range(64):
            chunk = mb_ref[pl.ds(8 * r, 8), :]
            acc1 = acc1 + (chunk >= m1).astype(jnp.int32)
            acc2 = acc2 + (chunk >= m2).astype(jnp.int32)
            acc3 = acc3 + (chunk >= m3).astype(jnp.int32)
        c1 = jnp.sum(acc1, axis=(0, 1), keepdims=True)
        c2 = jnp.sum(acc2, axis=(0, 1), keepdims=True)
        c3 = jnp.sum(acc3, axis=(0, 1), keepdims=True)
        ok1 = c1 >= neg_num
        ok2 = c2 >= neg_num
        ok3 = c3 >= neg_num
        lo2 = jnp.where(ok1, m1, lo)
        lo2 = jnp.where(ok2, m2, lo2)
        lo2 = jnp.where(ok3, m3, lo2)
        hi2 = jnp.where(jnp.logical_not(ok3), m3 - 1, hi)
        hi2 = jnp.where(jnp.logical_not(ok2), m2 - 1, hi2)
        hi2 = jnp.where(jnp.logical_not(ok1), m1 - 1, hi2)
        return lo2, hi2

    lo, _ = lax.fori_loop(
        0, 15, step,
        (jnp.zeros((1, 1), jnp.int32), jnp.full((1, 1), _ONE_BITS, jnp.int32)),
        unroll=5,
    )

    sel = ((bits >= lo) | pos) & effg
    cond = (pos_num == 0) | (neg_num == 0)
    sel_f = jnp.where(
        cond,
        (eff != 0.0).astype(jnp.float32),
        sel.astype(jnp.float32),
    )

    t_f = pos.astype(jnp.float32) * sel_f
    p_f = pred * sel_f
    pg = jnp.sum(p_f * t_f)
    p2 = jnp.sum(p_f * p_f)
    g2 = jnp.sum(t_f * t_f)
    text_loss = 1.0 - (2.0 * pg + _SMOOTH) / (p2 + g2 + _SMOOTH)

    pred_k = jax.nn.sigmoid(pk_ref[0, 0])
    sel2 = ((pred > 0.5) & effg).astype(jnp.float32)
    tk_f = (tk_ref[0, 0] > 0.5).astype(jnp.float32) * sel2
    pk_f = pred_k * sel2
    pg2 = jnp.sum(pk_f * tk_f)
    p2b = jnp.sum(pk_f * pk_f)
    g2b = jnp.sum(tk_f * tk_f)
    kernel_loss = 1.0 - (2.0 * pg2 + _SMOOTH) / (p2b + g2b + _SMOOTH)

    text_ref[0, 0, :] = jnp.full((128,), text_loss, dtype=jnp.float32)
    kern_ref[0, 0, :] = jnp.full((128,), kernel_loss, dtype=jnp.float32)


def kernel(preds, targets, effective_maps):
    n, _, h, w = preds.shape
    img_spec = pl.BlockSpec((1, h, w), lambda i: (i, 0, 0))
    ch0_spec = pl.BlockSpec((1, 1, h, w), lambda i: (i, 0, 0, 0))
    ch1_spec = pl.BlockSpec((1, 1, h, w), lambda i: (i, 1, 0, 0))
    out_spec = pl.BlockSpec((1, 1, 128), lambda i: (i, 0, 0))
    text, kern = pl.pallas_call(
        _loss_body,
        grid=(n,),
        in_specs=[ch0_spec, ch0_spec, ch1_spec, ch1_spec, img_spec],
        out_specs=[out_spec, out_spec],
        out_shape=[
            jax.ShapeDtypeStruct((n, 1, 128), jnp.float32),
            jax.ShapeDtypeStruct((n, 1, 128), jnp.float32),
        ],
        scratch_shapes=[pltpu.VMEM((512, 512), jnp.int32)],
        compiler_params=pltpu.CompilerParams(
            dimension_semantics=("arbitrary",),
        ),
    )(preds, targets, preds, targets, effective_maps)
    return text[:, 0, 0], kern[:, 0, 0]


# speedup vs baseline: 3.5895x; 1.0030x over previous
import jax
import jax.numpy as jnp
from jax import lax
from jax.experimental import pallas as pl
from jax.experimental.pallas import tpu as pltpu

_OHEM_RATIO = 3.0
_SMOOTH = 1e-06
_ONE_BITS = 0x3F800000


def _loss_body(pt_ref, tt_ref, pk_ref, tk_ref, eff_ref, text_ref, kern_ref,
               mb_ref):
    logits_t = pt_ref[0, 0]
    tt = tt_ref[0, 0]
    eff = eff_ref[0]
    pred = jax.nn.sigmoid(logits_t)

    pos = tt > 0.5
    neg = jnp.logical_not(pos)
    effg = eff > 0.5

    pos_num = jnp.sum(
        jnp.where(pos & effg, 1, 0), axis=(0, 1), keepdims=True
    )
    neg_total = jnp.sum(jnp.where(neg, 1, 0), axis=(0, 1), keepdims=True)
    neg_num = jnp.minimum(
        pos_num.astype(jnp.float32) * _OHEM_RATIO,
        neg_total.astype(jnp.float32),
    ).astype(jnp.int32)

    bits = lax.bitcast_convert_type(pred, jnp.int32)
    mbits = jnp.where(neg, bits, -1)

    mb_ref[...] = mbits

    def step(_, lohi):
        lo, hi = lohi
        s = (hi - lo + 4) // 4
        m1 = lo + s
        m2 = lo + 2 * s
        m3 = lo + 3 * s
        acc1 = jnp.zeros((8, 512), jnp.int32)
        acc2 = jnp.zeros((8, 512), jnp.int32)
        acc3 = jnp.zeros((8, 512), jnp.int32)
        for r in range(64):
            chunk = mb_ref[pl.ds(8 * r, 8), :]
            acc1 = acc1 + (chunk >= m1).astype(jnp.int32)
            acc2 = acc2 + (chunk >= m2).astype(jnp.int32)
            acc3 = acc3 + (chunk >= m3).astype(jnp.int32)
        c1 = jnp.sum(acc1, axis=(0, 1), keepdims=True)
        c2 = jnp.sum(acc2, axis=(0, 1), keepdims=True)
        c3 = jnp.sum(acc3, axis=(0, 1), keepdims=True)
        ok1 = c1 >= neg_num
        ok2 = c2 >= neg_num
        ok3 = c3 >= neg_num
        lo2 = jnp.where(ok1, m1, lo)
        lo2 = jnp.where(ok2, m2, lo2)
        lo2 = jnp.where(ok3, m3, lo2)
        hi2 = jnp.where(jnp.logical_not(ok3), m3 - 1, hi)
        hi2 = jnp.where(jnp.logical_not(ok2), m2 - 1, hi2)
        hi2 = jnp.where(jnp.logical_not(ok1), m1 - 1, hi2)
        return lo2, hi2

    lo, _ = lax.fori_loop(
        0, 15, step,
        (jnp.zeros((1, 1), jnp.int32), jnp.full((1, 1), _ONE_BITS, jnp.int32)),
        unroll=False,
    )

    sel = ((bits >= lo) | pos) & effg
    cond = (pos_num == 0) | (neg_num == 0)
    sel_f = jnp.where(
        cond,
        (eff != 0.0).astype(jnp.float32),
        sel.astype(jnp.float32),
    )

    t_f = pos.astype(jnp.float32) * sel_f
    p_f = pred * sel_f
    pg = jnp.sum(p_f * t_f)
    p2 = jnp.sum(p_f * p_f)
    g2 = jnp.sum(t_f * t_f)
    text_loss = 1.0 - (2.0 * pg + _SMOOTH) / (p2 + g2 + _SMOOTH)

    pred_k = jax.nn.sigmoid(pk_ref[0, 0])
    sel2 = ((pred > 0.5) & effg).astype(jnp.float32)
    tk_f = (tk_ref[0, 0] > 0.5).astype(jnp.float32) * sel2
    pk_f = pred_k * sel2
    pg2 = jnp.sum(pk_f * tk_f)
    p2b = jnp.sum(pk_f * pk_f)
    g2b = jnp.sum(tk_f * tk_f)
    kernel_loss = 1.0 - (2.0 * pg2 + _SMOOTH) / (p2b + g2b + _SMOOTH)

    text_ref[0, 0, :] = jnp.full((128,), text_loss, dtype=jnp.float32)
    kern_ref[0, 0, :] = jnp.full((128,), kernel_loss, dtype=jnp.float32)


def kernel(preds, targets, effective_maps):
    n, _, h, w = preds.shape
    img_spec = pl.BlockSpec((1, h, w), lambda i: (i, 0, 0))
    ch0_spec = pl.BlockSpec((1, 1, h, w), lambda i: (i, 0, 0, 0))
    ch1_spec = pl.BlockSpec((1, 1, h, w), lambda i: (i, 1, 0, 0))
    out_spec = pl.BlockSpec((1, 1, 128), lambda i: (i, 0, 0))
    text, kern = pl.pallas_call(
        _loss_body,
        grid=(n,),
        in_specs=[ch0_spec, ch0_spec, ch1_spec, ch1_spec, img_spec],
        out_specs=[out_spec, out_spec],
        out_shape=[
            jax.ShapeDtypeStruct((n, 1, 128), jnp.float32),
            jax.ShapeDtypeStruct((n, 1, 128), jnp.float32),
        ],
        scratch_shapes=[pltpu.VMEM((512, 512), jnp.int32)],
        compiler_params=pltpu.CompilerParams(
            dimension_semantics=("parallel",),
        ),
    )(preds, targets, preds, targets, effective_maps)
    return text[:, 0, 0], kern[:, 0, 0]
